# Initial kernel scaffold; baseline (speedup 1.0000x reference)
import jax
import jax.numpy as jnp
from jax.experimental import pallas as pl


def kernel(z, edge_index, emb_table, W, b):
    raise NotImplementedError("write your pallas kernel here")


# speedup vs baseline: 11.0953x; 11.0953x over previous
import functools

import jax
import jax.numpy as jnp
from jax import lax
from jax.experimental import pallas as pl
from jax.experimental.pallas import tpu as pltpu
from jax.experimental.pallas import tpu_sc as plsc

H = 128
NCLS = 128
BLK = 128
NW = 32


def _t12_body(emb_ref, w_ref, b_ref, out_ref):
    emb = emb_ref[...]
    t1 = jnp.dot(emb, w_ref[:H, :], preferred_element_type=jnp.float32)
    t2 = jnp.dot(emb, w_ref[H:, :], preferred_element_type=jnp.float32)
    t1 = t1 + b_ref[...]
    out_ref[...] = t1[:, None, :] + t2[None, :, :]


def _build_t12(emb_table, W, b):
    out = pl.pallas_call(
        _t12_body,
        out_shape=jax.ShapeDtypeStruct((NCLS, NCLS, H), jnp.float32),
    )(emb_table, W, b.reshape(1, H))
    return out.reshape(NCLS * NCLS, H)


def _edge_body(nb, z_hbm, src_hbm, dst_hbm, t12_hbm, out_hbm,
               z_v, si_v, di_v, cc_v, rows_v, sem):
    cid = lax.axis_index("c")
    sid = lax.axis_index("s")
    wid = sid * 2 + cid
    pltpu.sync_copy(z_hbm, z_v)
    niter = (nb - wid + NW - 1) // NW

    def body(i, carry):
        blk = wid + i * NW
        pltpu.sync_copy(src_hbm.at[blk], si_v)
        pltpu.sync_copy(dst_hbm.at[blk], di_v)
        for j in range(BLK // 16):
            s = plsc.load_gather(z_v, [si_v[pl.ds(j * 16, 16)]])
            d = plsc.load_gather(z_v, [di_v[pl.ds(j * 16, 16)]])
            cc_v[pl.ds(j * 16, 16)] = s * NCLS + d
        pltpu.async_copy(t12_hbm.at[cc_v], rows_v, sem).wait()
        pltpu.sync_copy(rows_v, out_hbm.at[pl.ds(blk * BLK, BLK)])
        return carry

    lax.fori_loop(0, niter, body, 0)


def _edge_call(nb, z, src, dst, t12):
    mesh = plsc.VectorSubcoreMesh(core_axis_name="c", subcore_axis_name="s")
    n_nodes = z.shape[0]
    fn = pl.kernel(
        functools.partial(_edge_body, nb),
        out_type=jax.ShapeDtypeStruct((nb * BLK, H), jnp.float32),
        mesh=mesh,
        scratch_types=[
            pltpu.VMEM((n_nodes,), jnp.int32),
            pltpu.VMEM((BLK,), jnp.int32),
            pltpu.VMEM((BLK,), jnp.int32),
            pltpu.VMEM((BLK,), jnp.int32),
            pltpu.VMEM((BLK, H), jnp.float32),
            pltpu.SemaphoreType.DMA,
        ],
        compiler_params=pltpu.CompilerParams(needs_layout_passes=False),
    )
    return fn(z, src, dst, t12)


def kernel(z, edge_index, emb_table, W, b):
    t12 = _build_t12(emb_table, W, b)
    n_edges = edge_index.shape[1]
    nb = n_edges // BLK
    src = edge_index[0].reshape(nb, BLK).astype(jnp.int32)
    dst = edge_index[1].reshape(nb, BLK).astype(jnp.int32)
    out = _edge_call(nb, z.astype(jnp.int32), src, dst, t12)
    return out[:, :, None, None]


# speedup vs baseline: 19.8764x; 1.7914x over previous
import functools

import jax
import jax.numpy as jnp
from jax import lax
from jax.experimental import pallas as pl
from jax.experimental.pallas import tpu as pltpu
from jax.experimental.pallas import tpu_sc as plsc

H = 128
NCLS = 128
BLK = 400
NW = 32
NITER = 25


def _t12_body(emb_ref, w_ref, b_ref, out_ref):
    emb = emb_ref[...]
    t1 = jnp.dot(emb, w_ref[:H, :], preferred_element_type=jnp.float32)
    t2 = jnp.dot(emb, w_ref[H:, :], preferred_element_type=jnp.float32)
    t1 = t1 + b_ref[...]
    out_ref[...] = t1[:, None, :] + t2[None, :, :]


def _build_t12(emb_table, W, b):
    out = pl.pallas_call(
        _t12_body,
        out_shape=jax.ShapeDtypeStruct((NCLS, NCLS, H), jnp.float32),
    )(emb_table, W, b.reshape(1, H))
    return out.reshape(NCLS * NCLS, H)


def _edge_body(z_hbm, sd_hbm, t12_hbm, out_hbm,
               z_v, sd_v0, sd_v1, cc_v0, cc_v1, rows_v0, rows_v1,
               isem0, isem1, gsem0, gsem1, ssem0, ssem1):
    cid = lax.axis_index("c")
    sid = lax.axis_index("s")
    wid = sid * 2 + cid
    base = wid * NITER

    sd_v = (sd_v0, sd_v1)
    cc_v = (cc_v0, cc_v1)
    rows_v = (rows_v0, rows_v1)
    isem = (isem0, isem1)
    gsem = (gsem0, gsem1)
    ssem = (ssem0, ssem1)

    pltpu.sync_copy(z_hbm, z_v)

    def idx_start(i, b):
        pltpu.async_copy(sd_hbm.at[base + i], sd_v[b], isem[b])

    def gather_start(i, b, prefetch_idx=True, wait_store=True):
        pltpu.make_async_copy(sd_hbm.at[0], sd_v[b], isem[b]).wait()
        for j in range(BLK // 16):
            s = plsc.load_gather(z_v, [sd_v[b][pl.ds(j * 16, 16)]])
            d = plsc.load_gather(z_v, [sd_v[b][pl.ds(BLK + j * 16, 16)]])
            cc_v[b][pl.ds(j * 16, 16)] = s * NCLS + d
        if prefetch_idx:
            idx_start(i + 2, b)
        if wait_store:
            pltpu.make_async_copy(out_hbm.at[pl.ds(0, BLK)], rows_v[b],
                                  ssem[b]).wait()
        pltpu.async_copy(t12_hbm.at[cc_v[b]], rows_v[b], gsem[b])

    def finish(i, b):
        pltpu.make_async_copy(t12_hbm.at[cc_v[b]], rows_v[b], gsem[b]).wait()
        pltpu.async_copy(rows_v[b], out_hbm.at[pl.ds((base + i) * BLK, BLK)],
                         ssem[b])

    idx_start(0, 0)
    idx_start(1, 1)
    gather_start(0, 0, wait_store=False)
    gather_start(1, 1, wait_store=False)

    def pair(g, carry):
        i0 = 2 * g
        finish(i0, 0)
        gather_start(i0 + 2, 0)
        finish(i0 + 1, 1)
        gather_start(i0 + 3, 1)
        return carry

    lax.fori_loop(0, (NITER - 5) // 2, pair, 0)

    finish(NITER - 5, 0)
    gather_start(NITER - 3, 0)
    finish(NITER - 4, 1)
    gather_start(NITER - 2, 1, prefetch_idx=False)
    finish(NITER - 3, 0)
    gather_start(NITER - 1, 0, prefetch_idx=False)
    finish(NITER - 2, 1)
    finish(NITER - 1, 0)

    pltpu.make_async_copy(out_hbm.at[pl.ds(0, BLK)], rows_v0, ssem0).wait()
    pltpu.make_async_copy(out_hbm.at[pl.ds(0, BLK)], rows_v1, ssem1).wait()


def _edge_call(z, sd, t12):
    mesh = plsc.VectorSubcoreMesh(core_axis_name="c", subcore_axis_name="s")
    n_nodes = z.shape[0]
    fn = pl.kernel(
        _edge_body,
        out_type=jax.ShapeDtypeStruct((NW * NITER * BLK, H), jnp.float32),
        mesh=mesh,
        scratch_types=[
            pltpu.VMEM((n_nodes,), jnp.int32),
            pltpu.VMEM((2 * BLK,), jnp.int32),
            pltpu.VMEM((2 * BLK,), jnp.int32),
            pltpu.VMEM((BLK,), jnp.int32),
            pltpu.VMEM((BLK,), jnp.int32),
            pltpu.VMEM((BLK, H), jnp.float32),
            pltpu.VMEM((BLK, H), jnp.float32),
            pltpu.SemaphoreType.DMA,
            pltpu.SemaphoreType.DMA,
            pltpu.SemaphoreType.DMA,
            pltpu.SemaphoreType.DMA,
            pltpu.SemaphoreType.DMA,
            pltpu.SemaphoreType.DMA,
        ],
        compiler_params=pltpu.CompilerParams(needs_layout_passes=False),
    )
    return fn(z, sd, t12)


def kernel(z, edge_index, emb_table, W, b):
    t12 = _build_t12(emb_table, W, b)
    n_edges = edge_index.shape[1]
    nb = n_edges // BLK
    src = edge_index[0].reshape(nb, BLK).astype(jnp.int32)
    dst = edge_index[1].reshape(nb, BLK).astype(jnp.int32)
    sd = jnp.concatenate([src, dst], axis=1)
    out = _edge_call(z.astype(jnp.int32), sd, t12)
    return out[:, :, None, None]


# speedup vs baseline: 20.1529x; 1.0139x over previous
import functools

import jax
import jax.numpy as jnp
from jax import lax
from jax.experimental import pallas as pl
from jax.experimental.pallas import tpu as pltpu
from jax.experimental.pallas import tpu_sc as plsc

H = 128
NCLS = 128
BLK = 400
NW = 32
NITER = 25


def _t12_body(emb_ref, w_ref, b_ref, out_ref):
    emb = emb_ref[...]
    t1 = jnp.dot(emb, w_ref[:H, :], preferred_element_type=jnp.float32)
    t2 = jnp.dot(emb, w_ref[H:, :], preferred_element_type=jnp.float32)
    t1 = t1 + b_ref[...]
    out_ref[...] = t1[:, None, :] + t2[None, :, :]


def _build_t12(emb_table, W, b):
    out = pl.pallas_call(
        _t12_body,
        out_shape=jax.ShapeDtypeStruct((NCLS, NCLS, H), jnp.float32),
    )(emb_table, W, b.reshape(1, H))
    return out.reshape(NCLS * NCLS, H)


def _edge_body(z_hbm, src_hbm, dst_hbm, t12_hbm, out_hbm,
               z_v, si_v0, si_v1, di_v0, di_v1, cc_v0, cc_v1,
               rows_v0, rows_v1,
               isem0, isem1, gsem0, gsem1, ssem0, ssem1):
    cid = lax.axis_index("c")
    sid = lax.axis_index("s")
    wid = sid * 2 + cid
    base = wid * NITER

    si_v = (si_v0, si_v1)
    di_v = (di_v0, di_v1)
    cc_v = (cc_v0, cc_v1)
    rows_v = (rows_v0, rows_v1)
    isem = (isem0, isem1)
    gsem = (gsem0, gsem1)
    ssem = (ssem0, ssem1)

    pltpu.sync_copy(z_hbm, z_v)

    def idx_start(i, b):
        pltpu.async_copy(src_hbm.at[base + i], si_v[b], isem[b])
        pltpu.async_copy(dst_hbm.at[base + i], di_v[b], isem[b])

    def gather_start(i, b, prefetch_idx=True, wait_store=True):
        pltpu.make_async_copy(src_hbm.at[0], si_v[b], isem[b]).wait()
        pltpu.make_async_copy(dst_hbm.at[0], di_v[b], isem[b]).wait()
        for j in range(BLK // 16):
            s = plsc.load_gather(z_v, [si_v[b][pl.ds(j * 16, 16)]])
            d = plsc.load_gather(z_v, [di_v[b][pl.ds(j * 16, 16)]])
            cc_v[b][pl.ds(j * 16, 16)] = s * NCLS + d
        if prefetch_idx:
            idx_start(i + 2, b)
        if wait_store:
            pltpu.make_async_copy(out_hbm.at[pl.ds(0, BLK)], rows_v[b],
                                  ssem[b]).wait()
        pltpu.async_copy(t12_hbm.at[cc_v[b]], rows_v[b], gsem[b])

    def finish(i, b):
        pltpu.make_async_copy(t12_hbm.at[cc_v[b]], rows_v[b], gsem[b]).wait()
        pltpu.async_copy(rows_v[b], out_hbm.at[pl.ds((base + i) * BLK, BLK)],
                         ssem[b])

    idx_start(0, 0)
    idx_start(1, 1)
    gather_start(0, 0, wait_store=False)
    gather_start(1, 1, wait_store=False)

    def pair(g, carry):
        i0 = 2 * g
        finish(i0, 0)
        gather_start(i0 + 2, 0)
        finish(i0 + 1, 1)
        gather_start(i0 + 3, 1)
        return carry

    lax.fori_loop(0, (NITER - 5) // 2, pair, 0)

    finish(NITER - 5, 0)
    gather_start(NITER - 3, 0)
    finish(NITER - 4, 1)
    gather_start(NITER - 2, 1, prefetch_idx=False)
    finish(NITER - 3, 0)
    gather_start(NITER - 1, 0, prefetch_idx=False)
    finish(NITER - 2, 1)
    finish(NITER - 1, 0)

    pltpu.make_async_copy(out_hbm.at[pl.ds(0, BLK)], rows_v0, ssem0).wait()
    pltpu.make_async_copy(out_hbm.at[pl.ds(0, BLK)], rows_v1, ssem1).wait()


def _edge_call(z, src, dst, t12):
    mesh = plsc.VectorSubcoreMesh(core_axis_name="c", subcore_axis_name="s")
    n_nodes = z.shape[0]
    fn = pl.kernel(
        _edge_body,
        out_type=jax.ShapeDtypeStruct((NW * NITER * BLK, H), jnp.float32),
        mesh=mesh,
        scratch_types=[
            pltpu.VMEM((n_nodes,), jnp.int32),
            pltpu.VMEM((BLK,), jnp.int32),
            pltpu.VMEM((BLK,), jnp.int32),
            pltpu.VMEM((BLK,), jnp.int32),
            pltpu.VMEM((BLK,), jnp.int32),
            pltpu.VMEM((BLK,), jnp.int32),
            pltpu.VMEM((BLK,), jnp.int32),
            pltpu.VMEM((BLK, H), jnp.float32),
            pltpu.VMEM((BLK, H), jnp.float32),
            pltpu.SemaphoreType.DMA,
            pltpu.SemaphoreType.DMA,
            pltpu.SemaphoreType.DMA,
            pltpu.SemaphoreType.DMA,
            pltpu.SemaphoreType.DMA,
            pltpu.SemaphoreType.DMA,
        ],
        compiler_params=pltpu.CompilerParams(needs_layout_passes=False),
    )
    return fn(z, src, dst, t12)


def kernel(z, edge_index, emb_table, W, b):
    t12 = _build_t12(emb_table, W, b)
    n_edges = edge_index.shape[1]
    nb = n_edges // BLK
    src = edge_index[0].reshape(nb, BLK).astype(jnp.int32)
    dst = edge_index[1].reshape(nb, BLK).astype(jnp.int32)
    out = _edge_call(z.astype(jnp.int32), src, dst, t12)
    return out[:, :, None, None]


# speedup vs baseline: 21.8775x; 1.0856x over previous
import functools

import jax
import jax.numpy as jnp
from jax import lax
from jax.experimental import pallas as pl
from jax.experimental.pallas import tpu as pltpu
from jax.experimental.pallas import tpu_sc as plsc

H = 128
NCLS = 128
BLK = 400
NW = 32
NITER = 25


def _t12_body(emb_ref, w_ref, b_ref, out_ref):
    emb = emb_ref[...]
    t1 = jnp.dot(emb, w_ref[:H, :], preferred_element_type=jnp.float32)
    t2 = jnp.dot(emb, w_ref[H:, :], preferred_element_type=jnp.float32)
    t1 = t1 + b_ref[...]
    out_ref[...] = t1[:, None, :] + t2[None, :, :]


def _build_t12(emb_table, W, b):
    out = pl.pallas_call(
        _t12_body,
        out_shape=jax.ShapeDtypeStruct((NCLS, NCLS, H), jnp.float32),
    )(emb_table, W, b.reshape(1, H))
    return out.reshape(NCLS * NCLS, H)


def _edge_body(z_hbm, ei_hbm, t12_hbm, out_hbm,
               z_v, si_v0, si_v1, di_v0, di_v1, cc_v0, cc_v1,
               rows_v0, rows_v1,
               isem0, isem1, gsem0, gsem1, ssem0, ssem1):
    cid = lax.axis_index("c")
    sid = lax.axis_index("s")
    wid = sid * 2 + cid
    base = wid * NITER

    si_v = (si_v0, si_v1)
    di_v = (di_v0, di_v1)
    cc_v = (cc_v0, cc_v1)
    rows_v = (rows_v0, rows_v1)
    isem = (isem0, isem1)
    gsem = (gsem0, gsem1)
    ssem = (ssem0, ssem1)

    pltpu.sync_copy(z_hbm, z_v)

    n_edges = NW * NITER * BLK

    def idx_start(i, b):
        st = (base + i) * BLK
        pltpu.async_copy(ei_hbm.at[pl.ds(st, BLK)], si_v[b], isem[b])
        pltpu.async_copy(ei_hbm.at[pl.ds(n_edges + st, BLK)], di_v[b],
                         isem[b])

    def gather_start(i, b, prefetch_idx=True, wait_store=True):
        pltpu.make_async_copy(ei_hbm.at[pl.ds(0, BLK)], si_v[b],
                              isem[b]).wait()
        pltpu.make_async_copy(ei_hbm.at[pl.ds(0, BLK)], di_v[b],
                              isem[b]).wait()
        for j in range(BLK // 16):
            s = plsc.load_gather(z_v, [si_v[b][pl.ds(j * 16, 16)]])
            d = plsc.load_gather(z_v, [di_v[b][pl.ds(j * 16, 16)]])
            cc_v[b][pl.ds(j * 16, 16)] = s * NCLS + d
        if prefetch_idx:
            idx_start(i + 2, b)
        if wait_store:
            pltpu.make_async_copy(out_hbm.at[pl.ds(0, BLK)], rows_v[b],
                                  ssem[b]).wait()
        pltpu.async_copy(t12_hbm.at[cc_v[b]], rows_v[b], gsem[b])

    def finish(i, b):
        pltpu.make_async_copy(t12_hbm.at[cc_v[b]], rows_v[b], gsem[b]).wait()
        pltpu.async_copy(rows_v[b], out_hbm.at[pl.ds((base + i) * BLK, BLK)],
                         ssem[b])

    idx_start(0, 0)
    idx_start(1, 1)
    gather_start(0, 0, wait_store=False)
    gather_start(1, 1, wait_store=False)

    def pair(g, carry):
        i0 = 2 * g
        finish(i0, 0)
        gather_start(i0 + 2, 0)
        finish(i0 + 1, 1)
        gather_start(i0 + 3, 1)
        return carry

    lax.fori_loop(0, (NITER - 5) // 2, pair, 0)

    finish(NITER - 5, 0)
    gather_start(NITER - 3, 0)
    finish(NITER - 4, 1)
    gather_start(NITER - 2, 1, prefetch_idx=False)
    finish(NITER - 3, 0)
    gather_start(NITER - 1, 0, prefetch_idx=False)
    finish(NITER - 2, 1)
    finish(NITER - 1, 0)

    pltpu.make_async_copy(out_hbm.at[pl.ds(0, BLK)], rows_v0, ssem0).wait()
    pltpu.make_async_copy(out_hbm.at[pl.ds(0, BLK)], rows_v1, ssem1).wait()


def _edge_call(z, ei, t12):
    mesh = plsc.VectorSubcoreMesh(core_axis_name="c", subcore_axis_name="s")
    n_nodes = z.shape[0]
    fn = pl.kernel(
        _edge_body,
        out_type=jax.ShapeDtypeStruct((NW * NITER * BLK, H), jnp.float32),
        mesh=mesh,
        scratch_types=[
            pltpu.VMEM((n_nodes,), jnp.int32),
            pltpu.VMEM((BLK,), jnp.int32),
            pltpu.VMEM((BLK,), jnp.int32),
            pltpu.VMEM((BLK,), jnp.int32),
            pltpu.VMEM((BLK,), jnp.int32),
            pltpu.VMEM((BLK,), jnp.int32),
            pltpu.VMEM((BLK,), jnp.int32),
            pltpu.VMEM((BLK, H), jnp.float32),
            pltpu.VMEM((BLK, H), jnp.float32),
            pltpu.SemaphoreType.DMA,
            pltpu.SemaphoreType.DMA,
            pltpu.SemaphoreType.DMA,
            pltpu.SemaphoreType.DMA,
            pltpu.SemaphoreType.DMA,
            pltpu.SemaphoreType.DMA,
        ],
        compiler_params=pltpu.CompilerParams(needs_layout_passes=False),
    )
    return fn(z, ei, t12)


def kernel(z, edge_index, emb_table, W, b):
    t12 = _build_t12(emb_table, W, b)
    ei_flat = edge_index.astype(jnp.int32).reshape(-1)
    out = _edge_call(z.astype(jnp.int32), ei_flat, t12)
    return out[:, :, None, None]


# speedup vs baseline: 23.4239x; 1.0707x over previous
import functools

import jax
import jax.numpy as jnp
from jax import lax
from jax.experimental import pallas as pl
from jax.experimental.pallas import tpu as pltpu
from jax.experimental.pallas import tpu_sc as plsc

H = 128
NCLS = 128
BLK = 400
NW = 32
NITER = 25


def _t12_body(emb_ref, w_ref, b_ref, out_ref):
    emb = emb_ref[...]
    t1 = jnp.dot(emb, w_ref[:H, :], preferred_element_type=jnp.float32)
    t2 = jnp.dot(emb, w_ref[H:, :], preferred_element_type=jnp.float32)
    t1 = t1 + b_ref[...]
    out_ref[...] = t1[:, None, :] + t2[None, :, :]


def _build_t12(emb_table, W, b):
    out = pl.pallas_call(
        _t12_body,
        out_shape=jax.ShapeDtypeStruct((NCLS, NCLS, H), jnp.float32),
    )(emb_table, W, b.reshape(1, H))
    return out.reshape(NCLS * NCLS, H)


def _edge_body(z_hbm, ei_hbm, t12_hbm, out_hbm,
               z_v, sd_v0, sd_v1, cc_v0, cc_v1,
               rows_v0, rows_v1,
               isem0, isem1, gsem0, gsem1, ssem0, ssem1):
    cid = lax.axis_index("c")
    sid = lax.axis_index("s")
    wid = sid * 2 + cid
    base = wid * NITER

    sd_v = (sd_v0, sd_v1)
    cc_v = (cc_v0, cc_v1)
    rows_v = (rows_v0, rows_v1)
    isem = (isem0, isem1)
    gsem = (gsem0, gsem1)
    ssem = (ssem0, ssem1)

    pltpu.sync_copy(z_hbm, z_v)

    WIN = 512

    def idx_start(i, b):
        st = (base + i) * BLK
        st_al = (st // 128) * 128
        pltpu.async_copy(ei_hbm.at[:, pl.ds(st_al, WIN)], sd_v[b], isem[b])

    def gather_start(i, b, prefetch_idx=True, wait_store=True):
        pltpu.make_async_copy(ei_hbm.at[:, pl.ds(0, WIN)], sd_v[b],
                              isem[b]).wait()
        st = (base + i) * BLK
        off = st - (st // 128) * 128
        for j in range(BLK // 16):
            s = plsc.load_gather(z_v, [sd_v[b][0, pl.ds(off + j * 16, 16)]])
            d = plsc.load_gather(z_v, [sd_v[b][1, pl.ds(off + j * 16, 16)]])
            cc_v[b][pl.ds(j * 16, 16)] = s * NCLS + d
        if prefetch_idx:
            idx_start(i + 2, b)
        if wait_store:
            pltpu.make_async_copy(out_hbm.at[pl.ds(0, BLK)], rows_v[b],
                                  ssem[b]).wait()
        pltpu.async_copy(t12_hbm.at[cc_v[b]], rows_v[b], gsem[b])

    def finish(i, b):
        pltpu.make_async_copy(t12_hbm.at[cc_v[b]], rows_v[b], gsem[b]).wait()
        pltpu.async_copy(rows_v[b], out_hbm.at[pl.ds((base + i) * BLK, BLK)],
                         ssem[b])

    idx_start(0, 0)
    idx_start(1, 1)
    gather_start(0, 0, wait_store=False)
    gather_start(1, 1, wait_store=False)

    def pair(g, carry):
        i0 = 2 * g
        finish(i0, 0)
        gather_start(i0 + 2, 0)
        finish(i0 + 1, 1)
        gather_start(i0 + 3, 1)
        return carry

    lax.fori_loop(0, (NITER - 5) // 2, pair, 0)

    finish(NITER - 5, 0)
    gather_start(NITER - 3, 0)
    finish(NITER - 4, 1)
    gather_start(NITER - 2, 1, prefetch_idx=False)
    finish(NITER - 3, 0)
    gather_start(NITER - 1, 0, prefetch_idx=False)
    finish(NITER - 2, 1)
    finish(NITER - 1, 0)

    pltpu.make_async_copy(out_hbm.at[pl.ds(0, BLK)], rows_v0, ssem0).wait()
    pltpu.make_async_copy(out_hbm.at[pl.ds(0, BLK)], rows_v1, ssem1).wait()


def _edge_call(z, ei, t12):
    mesh = plsc.VectorSubcoreMesh(core_axis_name="c", subcore_axis_name="s")
    n_nodes = z.shape[0]
    fn = pl.kernel(
        _edge_body,
        out_type=jax.ShapeDtypeStruct((NW * NITER * BLK, H), jnp.float32),
        mesh=mesh,
        scratch_types=[
            pltpu.VMEM((n_nodes,), jnp.int32),
            pltpu.VMEM((2, 512), jnp.int32),
            pltpu.VMEM((2, 512), jnp.int32),
            pltpu.VMEM((BLK,), jnp.int32),
            pltpu.VMEM((BLK,), jnp.int32),
            pltpu.VMEM((BLK, H), jnp.float32),
            pltpu.VMEM((BLK, H), jnp.float32),
            pltpu.SemaphoreType.DMA,
            pltpu.SemaphoreType.DMA,
            pltpu.SemaphoreType.DMA,
            pltpu.SemaphoreType.DMA,
            pltpu.SemaphoreType.DMA,
            pltpu.SemaphoreType.DMA,
        ],
        compiler_params=pltpu.CompilerParams(needs_layout_passes=False),
    )
    return fn(z, ei, t12)


def kernel(z, edge_index, emb_table, W, b):
    t12 = _build_t12(emb_table, W, b)
    out = _edge_call(z.astype(jnp.int32), edge_index.astype(jnp.int32), t12)
    return out[:, :, None, None]


# speedup vs baseline: 23.6390x; 1.0092x over previous
import jax
import jax.numpy as jnp
from jax import lax
from jax.experimental import pallas as pl
from jax.experimental.pallas import tpu as pltpu
from jax.experimental.pallas import tpu_sc as plsc

H = 128
NCLS = 128
BLK = 80
NW = 32
NITER = 125
NBUF = 5
WIN = 256


def _t12_body(emb_ref, w_ref, b_ref, out_ref):
    emb = emb_ref[...]
    t1 = jnp.dot(emb, w_ref[:H, :], preferred_element_type=jnp.float32)
    t2 = jnp.dot(emb, w_ref[H:, :], preferred_element_type=jnp.float32)
    t1 = t1 + b_ref[...]
    out_ref[...] = t1[:, None, :] + t2[None, :, :]


def _build_t12(emb_table, W, b):
    out = pl.pallas_call(
        _t12_body,
        out_shape=jax.ShapeDtypeStruct((NCLS, NCLS, H), jnp.float32),
    )(emb_table, W, b.reshape(1, H))
    return out.reshape(NCLS * NCLS, H)


def _edge_body(z_hbm, ei_hbm, t12_hbm, out_hbm, *scr):
    z_v = scr[0]
    sd_v = scr[1:1 + NBUF]
    cc_v = scr[1 + NBUF:1 + 2 * NBUF]
    rows_v = scr[1 + 2 * NBUF:1 + 3 * NBUF]
    isem = scr[1 + 3 * NBUF:1 + 4 * NBUF]
    gsem = scr[1 + 4 * NBUF:1 + 5 * NBUF]
    ssem = scr[1 + 5 * NBUF:1 + 6 * NBUF]

    cid = lax.axis_index("c")
    sid = lax.axis_index("s")
    wid = sid * 2 + cid
    base = wid * NITER

    pltpu.sync_copy(z_hbm, z_v)

    def idx_start(i, b):
        st = (base + i) * BLK
        st_al = (st // 128) * 128
        pltpu.async_copy(ei_hbm.at[:, pl.ds(st_al, WIN)], sd_v[b], isem[b])

    def gather_start(i, b, prefetch_idx=True, wait_store=True):
        pltpu.make_async_copy(ei_hbm.at[:, pl.ds(0, WIN)], sd_v[b],
                              isem[b]).wait()
        st = (base + i) * BLK
        off = st - (st // 128) * 128
        for j in range(BLK // 16):
            s = plsc.load_gather(z_v, [sd_v[b][0, pl.ds(off + j * 16, 16)]])
            d = plsc.load_gather(z_v, [sd_v[b][1, pl.ds(off + j * 16, 16)]])
            cc_v[b][pl.ds(j * 16, 16)] = s * NCLS + d
        if prefetch_idx:
            idx_start(i + NBUF, b)
        if wait_store:
            pltpu.make_async_copy(out_hbm.at[pl.ds(0, BLK)], rows_v[b],
                                  ssem[b]).wait()
        pltpu.async_copy(t12_hbm.at[cc_v[b]], rows_v[b], gsem[b])

    def finish(i, b):
        pltpu.make_async_copy(t12_hbm.at[cc_v[b]], rows_v[b], gsem[b]).wait()
        pltpu.async_copy(rows_v[b], out_hbm.at[pl.ds((base + i) * BLK, BLK)],
                         ssem[b])

    for b in range(NBUF):
        idx_start(b, b)
    for b in range(NBUF):
        gather_start(b, b, wait_store=False)

    def group(g, carry):
        i0 = NBUF * g
        for b in range(NBUF):
            finish(i0 + b, b)
            gather_start(i0 + b + NBUF, b)
        return carry

    lax.fori_loop(0, NITER // NBUF - 2, group, 0)

    for b in range(NBUF):
        finish(NITER - 2 * NBUF + b, b)
        gather_start(NITER - NBUF + b, b, prefetch_idx=False)
    for b in range(NBUF):
        finish(NITER - NBUF + b, b)

    for b in range(NBUF):
        pltpu.make_async_copy(out_hbm.at[pl.ds(0, BLK)], rows_v[b],
                              ssem[b]).wait()


def _edge_call(z, ei, t12):
    mesh = plsc.VectorSubcoreMesh(core_axis_name="c", subcore_axis_name="s")
    n_nodes = z.shape[0]
    scratch = [pltpu.VMEM((n_nodes,), jnp.int32)]
    scratch += [pltpu.VMEM((2, WIN), jnp.int32) for _ in range(NBUF)]
    scratch += [pltpu.VMEM((BLK,), jnp.int32) for _ in range(NBUF)]
    scratch += [pltpu.VMEM((BLK, H), jnp.float32) for _ in range(NBUF)]
    scratch += [pltpu.SemaphoreType.DMA for _ in range(3 * NBUF)]
    fn = pl.kernel(
        _edge_body,
        out_type=jax.ShapeDtypeStruct((NW * NITER * BLK, H), jnp.float32),
        mesh=mesh,
        scratch_types=scratch,
        compiler_params=pltpu.CompilerParams(needs_layout_passes=False),
    )
    return fn(z, ei, t12)


def kernel(z, edge_index, emb_table, W, b):
    t12 = _build_t12(emb_table, W, b)
    out = _edge_call(z.astype(jnp.int32), edge_index.astype(jnp.int32), t12)
    return out[:, :, None, None]
